# baseline (speedup 1.0000x reference)
import functools

import jax
import jax.numpy as jnp
from jax import lax
from jax.experimental import pallas as pl
from jax.experimental.pallas import tpu as pltpu
from jax.experimental.pallas import tpu_sc as plsc

N_NODES = 10000
N_EDGES = 320000
D_FEAT = 128
D_OUT = 32
N_GRAPHS = 64

NC = 2
NS = 16
LANES = 16

EPW = N_EDGES // (NC * NS)
CHUNK = 128
NCHUNK = (EPW + CHUNK - 1) // CHUNK
PADDED = NCHUNK * CHUNK
WLEN = PADDED
TBL = N_NODES * N_GRAPHS
TPW = TBL // NS
ZB = 10000


def _sc_body(edge_hbm, pt_hbm, b_hbm, out_hbm,
             ev, e_v, bat_v, w_buf, f_buf, z_v, y_v, table,
             in_sem, zt_sem, sc_sem, fl_sem):
    c = lax.axis_index("c")
    s = lax.axis_index("s")
    wid = c * NS + s
    base = wid * EPW

    base_al = jnp.minimum((base // 128) * 128, N_EDGES - WLEN)
    off0 = base - base_al
    in_cps = [
        pltpu.async_copy(
            edge_hbm.at[:, pl.ds(pl.multiple_of(base_al, 128), WLEN)],
            ev, in_sem),
        pltpu.async_copy(pt_hbm.at[0], e_v, in_sem),
        pltpu.async_copy(b_hbm, bat_v, in_sem),
    ]

    with jax.named_scope("zero"):
        @plsc.parallel_loop(0, ZB // LANES, step=1, unroll=8)
        def zero_loop(i):
            z_v[pl.ds(pl.multiple_of(i * LANES, LANES), LANES)] = jnp.zeros(
                (LANES,), jnp.float32)
        z_cps = [
            pltpu.async_copy(z_v, table.at[pl.ds(s * TPW + k * ZB, ZB)],
                             zt_sem)
            for k in range(TPW // ZB)
        ]

        for k in range((PADDED - EPW) // LANES):
            w_buf[NCHUNK - 1, pl.ds(EPW % CHUNK + k * LANES, LANES)] = (
                jnp.zeros((LANES,), jnp.float32))
            f_buf[NCHUNK - 1, pl.ds(EPW % CHUNK + k * LANES, LANES)] = (
                jnp.zeros((LANES,), jnp.int32))

    with jax.named_scope("stage_wait"):
        for cp in in_cps:
            cp.wait()

    HALF = (EPW // LANES) // 2

    def edge_span(lo, hi):
        @plsc.parallel_loop(lo, hi, step=1, unroll=8)
        def edge_loop(i):
            off = pl.ds(pl.multiple_of(off0 + i * LANES, LANES), LANES)
            sn = ev[0, off]
            d = ev[1, off]
            w = plsc.load_gather(e_v, [d])
            g = plsc.load_gather(bat_v, [d])
            f = sn * N_GRAPHS + g
            row = i // (CHUNK // LANES)
            col = (i % (CHUNK // LANES)) * LANES
            w_buf[row, pl.ds(col, LANES)] = w
            f_buf[row, pl.ds(col, LANES)] = f

    ROWS1 = HALF // (CHUNK // LANES)
    with jax.named_scope("edges1"):
        edge_span(0, ROWS1 * (CHUNK // LANES))

    with jax.named_scope("zero_wait"):
        for cp in z_cps:
            cp.wait()
        plsc.subcore_barrier()

    with jax.named_scope("scatter1"):
        sc_cps = [
            pltpu.async_copy(w_buf.at[j], table.at[f_buf.at[j]], sc_sem,
                             add=True)
            for j in range(ROWS1)
        ]
    with jax.named_scope("edges2"):
        edge_span(ROWS1 * (CHUNK // LANES), EPW // LANES)
    with jax.named_scope("scatter2"):
        sc_cps += [
            pltpu.async_copy(w_buf.at[j], table.at[f_buf.at[j]], sc_sem,
                             add=True)
            for j in range(ROWS1, NCHUNK)
        ]
        for cp in sc_cps:
            cp.wait()

    plsc.subcore_barrier()

    bufs = [z_v, y_v]
    prev = None
    for k in range(TPW // ZB):
        buf = bufs[k % 2]
        pltpu.sync_copy(table.at[pl.ds(s * TPW + k * ZB, ZB)], buf)
        if prev is not None:
            prev.wait()
        prev = pltpu.async_copy(
            buf, out_hbm.at[pl.ds(c * TBL + s * TPW + k * ZB, ZB)], fl_sem)
    prev.wait()


@functools.partial(jax.jit, static_argnums=())
def _sc_build_table(edge_index, pt, bat):
    mesh = plsc.VectorSubcoreMesh(core_axis_name="c", subcore_axis_name="s")
    f = pl.kernel(
        _sc_body,
        out_type=jax.ShapeDtypeStruct((NC * TBL,), jnp.float32),
        mesh=mesh,
        compiler_params=pltpu.CompilerParams(needs_layout_passes=False),
        scratch_types=[
            pltpu.VMEM((2, WLEN), jnp.int32),
            pltpu.VMEM((N_NODES,), jnp.float32),
            pltpu.VMEM((N_NODES,), jnp.int32),
            pltpu.VMEM((NCHUNK, CHUNK), jnp.float32),
            pltpu.VMEM((NCHUNK, CHUNK), jnp.int32),
            pltpu.VMEM((ZB,), jnp.float32),
            pltpu.VMEM((ZB,), jnp.float32),
            pltpu.VMEM_SHARED((TBL,), jnp.float32),
            pltpu.SemaphoreType.DMA,
            pltpu.SemaphoreType.DMA,
            pltpu.SemaphoreType.DMA,
            pltpu.SemaphoreType.DMA,
        ],
    )
    return f(edge_index, pt, bat)


KB = 25
BK = (N_NODES // 2) // KB


def _tc_body(s_ref, x_ref, w_ref, b_ref, o_ref, acc, cacc):
    i = pl.program_id(0)
    st = s_ref[0] + s_ref[1]
    m = lax.dot_general(st, x_ref[...], (((0,), (0,)), ((), ())),
                        preferred_element_type=jnp.float32)
    cs = jnp.sum(st, axis=0, keepdims=True)

    @pl.when(i == 0)
    def _init():
        acc[...] = m
        cacc[...] = cs

    @pl.when(i > 0)
    def _accum():
        acc[...] += m
        cacc[...] += cs

    @pl.when(i == KB - 1)
    def _finish():
        mm = acc[...]
        g = mm[:N_GRAPHS, :D_FEAT] + mm[N_GRAPHS:, D_FEAT:]
        cnt128 = cacc[...]
        cnt = cnt128[:, :N_GRAPHS] + cnt128[:, N_GRAPHS:]
        o_ref[...] = (
            lax.dot_general(w_ref[...], g, (((0,), (1,)), ((), ())),
                            preferred_element_type=jnp.float32)
            + b_ref[...] * cnt)


def _tc_finish(s3, xr, w, b_col):
    ot = pl.pallas_call(
        _tc_body,
        grid=(KB,),
        in_specs=[
            pl.BlockSpec((NC, BK, 2 * N_GRAPHS), lambda i: (0, i, 0)),
            pl.BlockSpec((BK, 2 * D_FEAT), lambda i: (i, 0)),
            pl.BlockSpec((D_FEAT, D_OUT), lambda i: (0, 0)),
            pl.BlockSpec((D_OUT, 1), lambda i: (0, 0)),
        ],
        out_specs=pl.BlockSpec((D_OUT, N_GRAPHS), lambda i: (0, 0)),
        out_shape=jax.ShapeDtypeStruct((D_OUT, N_GRAPHS), jnp.float32),
        scratch_shapes=[
            pltpu.VMEM((2 * N_GRAPHS, 2 * D_FEAT), jnp.float32),
            pltpu.VMEM((1, 2 * N_GRAPHS), jnp.float32),
        ],
    )(s3, xr, w, b_col)
    return jnp.swapaxes(ot, 0, 1)


def kernel(x, p, edge_index, batch, W, b):
    pt = jnp.swapaxes(p, 0, 1)
    bat = batch.astype(jnp.int32)
    s_tbl = _sc_build_table(edge_index.astype(jnp.int32), pt, bat)
    s3 = s_tbl.reshape(NC, N_NODES // 2, 2 * N_GRAPHS)
    xr = x.reshape(N_NODES // 2, 2 * D_FEAT)
    return _tc_finish(s3, xr, W, b.reshape(D_OUT, 1))


# speedup vs baseline: 1.2560x; 1.2560x over previous
import functools

import jax
import jax.numpy as jnp
from jax import lax
from jax.experimental import pallas as pl
from jax.experimental.pallas import tpu as pltpu
from jax.experimental.pallas import tpu_sc as plsc

N_NODES = 10000
N_EDGES = 320000
D_FEAT = 128
D_OUT = 32
N_GRAPHS = 64

NC = 2
NS = 16
LANES = 16

EPW = N_EDGES // (NC * NS)
CHUNK = 128
NCHUNK = (EPW + CHUNK - 1) // CHUNK
PADDED = NCHUNK * CHUNK
WLEN = PADDED
TBL = N_NODES * N_GRAPHS
TPW = TBL // NS
ZB = 10000


W1 = 5120
W2 = WLEN - W1


def _sc_body(edge_hbm, eb_hbm, out_hbm,
             ev, eb_v, w_buf, f_buf, z_v, y_v, table,
             in_sem, w2_sem, zt_sem, sc_sem, fl_sem):
    c = lax.axis_index("c")
    s = lax.axis_index("s")
    wid = c * NS + s
    base = wid * EPW

    base_al = jnp.minimum((base // 128) * 128, N_EDGES - WLEN)
    off0 = base - base_al
    in_cps = [
        pltpu.async_copy(
            edge_hbm.at[:, pl.ds(pl.multiple_of(base_al, 128), W1)],
            ev.at[:, pl.ds(0, W1)], in_sem),
        pltpu.async_copy(eb_hbm, eb_v, in_sem),
    ]
    w2_cp = pltpu.async_copy(
        edge_hbm.at[:, pl.ds(pl.multiple_of(base_al + W1, 128), W2)],
        ev.at[:, pl.ds(W1, W2)], w2_sem)

    with jax.named_scope("zero"):
        @plsc.parallel_loop(0, ZB // LANES, step=1, unroll=8)
        def zero_loop(i):
            z_v[pl.ds(pl.multiple_of(i * LANES, LANES), LANES)] = jnp.zeros(
                (LANES,), jnp.float32)
        z_cps = [
            pltpu.async_copy(z_v, table.at[pl.ds(s * TPW + k * ZB, ZB)],
                             zt_sem)
            for k in range(TPW // ZB)
        ]

        for k in range((PADDED - EPW) // LANES):
            w_buf[NCHUNK - 1, pl.ds(EPW % CHUNK + k * LANES, LANES)] = (
                jnp.zeros((LANES,), jnp.float32))
            f_buf[NCHUNK - 1, pl.ds(EPW % CHUNK + k * LANES, LANES)] = (
                jnp.zeros((LANES,), jnp.int32))

    with jax.named_scope("stage_wait"):
        for cp in in_cps:
            cp.wait()

    HALF = (EPW // LANES) // 2

    def edge_span(lo, hi):
        @plsc.parallel_loop(lo, hi, step=1, unroll=8)
        def edge_loop(i):
            off = pl.ds(pl.multiple_of(off0 + i * LANES, LANES), LANES)
            sn = ev[0, off]
            d = ev[1, off]
            zeros16 = jnp.zeros((LANES,), jnp.int32)
            w = plsc.bitcast(
                plsc.load_gather(eb_v, [zeros16, d]), jnp.float32)
            g = plsc.load_gather(eb_v, [zeros16 + 1, d])
            f = sn * N_GRAPHS + g
            row = i // (CHUNK // LANES)
            col = (i % (CHUNK // LANES)) * LANES
            w_buf[row, pl.ds(col, LANES)] = w
            f_buf[row, pl.ds(col, LANES)] = f

    ROWS1 = HALF // (CHUNK // LANES)
    with jax.named_scope("edges1"):
        edge_span(0, ROWS1 * (CHUNK // LANES))

    with jax.named_scope("zero_wait"):
        for cp in z_cps:
            cp.wait()
        plsc.subcore_barrier()

    with jax.named_scope("scatter1"):
        sc_cps = [
            pltpu.async_copy(w_buf.at[j], table.at[f_buf.at[j]], sc_sem,
                             add=True)
            for j in range(ROWS1)
        ]
    with jax.named_scope("edges2"):
        w2_cp.wait()
        edge_span(ROWS1 * (CHUNK // LANES), EPW // LANES)
    with jax.named_scope("scatter2"):
        sc_cps += [
            pltpu.async_copy(w_buf.at[j], table.at[f_buf.at[j]], sc_sem,
                             add=True)
            for j in range(ROWS1, NCHUNK)
        ]
        for cp in sc_cps:
            cp.wait()

    plsc.subcore_barrier()

    bufs = [z_v, y_v]
    prev = None
    for k in range(TPW // ZB):
        buf = bufs[k % 2]
        pltpu.sync_copy(table.at[pl.ds(s * TPW + k * ZB, ZB)], buf)
        if prev is not None:
            prev.wait()
        prev = pltpu.async_copy(
            buf, out_hbm.at[pl.ds(c * TBL + s * TPW + k * ZB, ZB)], fl_sem)
    prev.wait()


@functools.partial(jax.jit, static_argnums=())
def _sc_build_table(edge_index, eb):
    mesh = plsc.VectorSubcoreMesh(core_axis_name="c", subcore_axis_name="s")
    f = pl.kernel(
        _sc_body,
        out_type=jax.ShapeDtypeStruct((NC * TBL,), jnp.float32),
        mesh=mesh,
        compiler_params=pltpu.CompilerParams(needs_layout_passes=False),
        scratch_types=[
            pltpu.VMEM((2, WLEN), jnp.int32),
            pltpu.VMEM((2, N_NODES), jnp.int32),
            pltpu.VMEM((NCHUNK, CHUNK), jnp.float32),
            pltpu.VMEM((NCHUNK, CHUNK), jnp.int32),
            pltpu.VMEM((ZB,), jnp.float32),
            pltpu.VMEM((ZB,), jnp.float32),
            pltpu.VMEM_SHARED((TBL,), jnp.float32),
            pltpu.SemaphoreType.DMA,
            pltpu.SemaphoreType.DMA,
            pltpu.SemaphoreType.DMA,
            pltpu.SemaphoreType.DMA,
            pltpu.SemaphoreType.DMA,
        ],
    )
    return f(edge_index, eb)


KB = 5
BK = (N_NODES // 2) // KB


def _tc_body(s_ref, x_ref, w_ref, b_ref, o_ref, acc, cacc):
    i = pl.program_id(0)
    st = s_ref[0] + s_ref[1]
    m = lax.dot_general(st, x_ref[...], (((0,), (0,)), ((), ())),
                        preferred_element_type=jnp.float32)
    cs = jnp.sum(st, axis=0, keepdims=True)

    @pl.when(i == 0)
    def _init():
        acc[...] = m
        cacc[...] = cs

    @pl.when(i > 0)
    def _accum():
        acc[...] += m
        cacc[...] += cs

    @pl.when(i == KB - 1)
    def _finish():
        mm = acc[...]
        g = mm[:N_GRAPHS, :D_FEAT] + mm[N_GRAPHS:, D_FEAT:]
        cnt128 = cacc[...]
        cnt = cnt128[:, :N_GRAPHS] + cnt128[:, N_GRAPHS:]
        o_ref[...] = (
            lax.dot_general(w_ref[...], g, (((0,), (1,)), ((), ())),
                            preferred_element_type=jnp.float32)
            + b_ref[...] * cnt)


def _tc_finish(s3, xr, w, b_col):
    ot = pl.pallas_call(
        _tc_body,
        grid=(KB,),
        in_specs=[
            pl.BlockSpec((NC, BK, 2 * N_GRAPHS), lambda i: (0, i, 0)),
            pl.BlockSpec((BK, 2 * D_FEAT), lambda i: (i, 0)),
            pl.BlockSpec((D_FEAT, D_OUT), lambda i: (0, 0)),
            pl.BlockSpec((D_OUT, 1), lambda i: (0, 0)),
        ],
        out_specs=pl.BlockSpec((D_OUT, N_GRAPHS), lambda i: (0, 0)),
        out_shape=jax.ShapeDtypeStruct((D_OUT, N_GRAPHS), jnp.float32),
        scratch_shapes=[
            pltpu.VMEM((2 * N_GRAPHS, 2 * D_FEAT), jnp.float32),
            pltpu.VMEM((1, 2 * N_GRAPHS), jnp.float32),
        ],
    )(s3, xr, w, b_col)
    return jnp.swapaxes(ot, 0, 1)


def kernel(x, p, edge_index, batch, W, b):
    pt = jnp.swapaxes(p, 0, 1)
    eb = jnp.concatenate(
        [lax.bitcast_convert_type(pt[0:1], jnp.int32),
         batch.astype(jnp.int32)[None]], axis=0)
    s_tbl = _sc_build_table(edge_index.astype(jnp.int32), eb)
    s3 = s_tbl.reshape(NC, N_NODES // 2, 2 * N_GRAPHS)
    xr = x.reshape(N_NODES // 2, 2 * D_FEAT)
    return _tc_finish(s3, xr, W, b.reshape(D_OUT, 1))


# speedup vs baseline: 1.2933x; 1.0297x over previous
import functools

import jax
import jax.numpy as jnp
from jax import lax
from jax.experimental import pallas as pl
from jax.experimental.pallas import tpu as pltpu
from jax.experimental.pallas import tpu_sc as plsc

N_NODES = 10000
N_EDGES = 320000
D_FEAT = 128
D_OUT = 32
N_GRAPHS = 64

NC = 2
NS = 16
LANES = 16

EPW = N_EDGES // (NC * NS)
CHUNK = 128
NCHUNK = (EPW + CHUNK - 1) // CHUNK
PADDED = NCHUNK * CHUNK
WLEN = PADDED
TBL = N_NODES * N_GRAPHS
TPW = TBL // NS
ZB = 10000


def _sc_body(edge_hbm, pt_hbm, b_hbm, out_hbm,
             ev, e_v, bat_v, w_buf, f_buf, z_v, y_v, table,
             in_sem, zt_sem, sc_sem, fl_sem):
    c = lax.axis_index("c")
    s = lax.axis_index("s")
    wid = c * NS + s
    base = wid * EPW

    base_al = jnp.minimum((base // 128) * 128, N_EDGES - WLEN)
    off0 = base - base_al
    in_cps = [
        pltpu.async_copy(
            edge_hbm.at[:, pl.ds(pl.multiple_of(base_al, 128), WLEN)],
            ev, in_sem),
        pltpu.async_copy(pt_hbm.at[0], e_v, in_sem),
        pltpu.async_copy(b_hbm, bat_v, in_sem),
    ]

    @plsc.parallel_loop(0, ZB // LANES, step=1, unroll=8)
    def zero_loop(i):
        z_v[pl.ds(pl.multiple_of(i * LANES, LANES), LANES)] = jnp.zeros(
            (LANES,), jnp.float32)

    z_cps = [
        pltpu.async_copy(z_v, table.at[pl.ds(s * TPW + k * ZB, ZB)], zt_sem)
        for k in range(TPW // ZB)
    ]

    for k in range((PADDED - EPW) // LANES):
        w_buf[NCHUNK - 1, pl.ds(EPW % CHUNK + k * LANES, LANES)] = jnp.zeros(
            (LANES,), jnp.float32)
        f_buf[NCHUNK - 1, pl.ds(EPW % CHUNK + k * LANES, LANES)] = jnp.zeros(
            (LANES,), jnp.int32)

    for cp in in_cps:
        cp.wait()

    def edge_span(lo, hi):
        @plsc.parallel_loop(lo, hi, step=1, unroll=8)
        def edge_loop(i):
            off = pl.ds(pl.multiple_of(off0 + i * LANES, LANES), LANES)
            sn = ev[0, off]
            d = ev[1, off]
            w = plsc.load_gather(e_v, [d])
            g = plsc.load_gather(bat_v, [d])
            f = sn * N_GRAPHS + g
            row = i // (CHUNK // LANES)
            col = (i % (CHUNK // LANES)) * LANES
            w_buf[row, pl.ds(col, LANES)] = w
            f_buf[row, pl.ds(col, LANES)] = f

    ROWS1 = (EPW // LANES) // 2 // (CHUNK // LANES)
    edge_span(0, ROWS1 * (CHUNK // LANES))

    for cp in z_cps:
        cp.wait()
    plsc.subcore_barrier()

    sc_cps = [
        pltpu.async_copy(w_buf.at[j], table.at[f_buf.at[j]], sc_sem,
                         add=True)
        for j in range(ROWS1)
    ]
    edge_span(ROWS1 * (CHUNK // LANES), EPW // LANES)
    sc_cps += [
        pltpu.async_copy(w_buf.at[j], table.at[f_buf.at[j]], sc_sem,
                         add=True)
        for j in range(ROWS1, NCHUNK)
    ]
    for cp in sc_cps:
        cp.wait()

    plsc.subcore_barrier()

    bufs = [z_v, y_v]
    prev = None
    for k in range(TPW // ZB):
        buf = bufs[k % 2]
        pltpu.sync_copy(table.at[pl.ds(s * TPW + k * ZB, ZB)], buf)
        if prev is not None:
            prev.wait()
        prev = pltpu.async_copy(
            buf, out_hbm.at[pl.ds(c * TBL + s * TPW + k * ZB, ZB)], fl_sem)
    prev.wait()


@functools.partial(jax.jit, static_argnums=())
def _sc_build_table(edge_index, pt, bat):
    mesh = plsc.VectorSubcoreMesh(core_axis_name="c", subcore_axis_name="s")
    f = pl.kernel(
        _sc_body,
        out_type=jax.ShapeDtypeStruct((NC * TBL,), jnp.float32),
        mesh=mesh,
        compiler_params=pltpu.CompilerParams(needs_layout_passes=False),
        scratch_types=[
            pltpu.VMEM((2, WLEN), jnp.int32),
            pltpu.VMEM((N_NODES,), jnp.float32),
            pltpu.VMEM((N_NODES,), jnp.int32),
            pltpu.VMEM((NCHUNK, CHUNK), jnp.float32),
            pltpu.VMEM((NCHUNK, CHUNK), jnp.int32),
            pltpu.VMEM((ZB,), jnp.float32),
            pltpu.VMEM((ZB,), jnp.float32),
            pltpu.VMEM_SHARED((TBL,), jnp.float32),
            pltpu.SemaphoreType.DMA,
            pltpu.SemaphoreType.DMA,
            pltpu.SemaphoreType.DMA,
            pltpu.SemaphoreType.DMA,
        ],
    )
    return f(edge_index, pt, bat)


def _tc_body(s_ref, x_ref, w_ref, b_ref, o_ref):
    st = s_ref[0] + s_ref[1]
    m = lax.dot_general(st, x_ref[...], (((0,), (0,)), ((), ())),
                        preferred_element_type=jnp.float32)
    g = m[:N_GRAPHS, :D_FEAT] + m[N_GRAPHS:, D_FEAT:]
    cnt128 = jnp.sum(st, axis=0, keepdims=True)
    cnt = cnt128[:, :N_GRAPHS] + cnt128[:, N_GRAPHS:]
    o_ref[...] = (
        lax.dot_general(w_ref[...], g, (((0,), (1,)), ((), ())),
                        preferred_element_type=jnp.float32)
        + b_ref[...] * cnt)


def _tc_finish(s3, xr, w, b_col):
    ot = pl.pallas_call(
        _tc_body,
        out_shape=jax.ShapeDtypeStruct((D_OUT, N_GRAPHS), jnp.float32),
    )(s3, xr, w, b_col)
    return jnp.swapaxes(ot, 0, 1)


def kernel(x, p, edge_index, batch, W, b):
    pt = jnp.swapaxes(p, 0, 1)
    bat = batch.astype(jnp.int32)
    s_tbl = _sc_build_table(edge_index.astype(jnp.int32), pt, bat)
    s3 = s_tbl.reshape(NC, N_NODES // 2, 2 * N_GRAPHS)
    xr = x.reshape(N_NODES // 2, 2 * D_FEAT)
    return _tc_finish(s3, xr, W, b.reshape(D_OUT, 1))
